# Initial kernel scaffold; baseline (speedup 1.0000x reference)
import functools

import jax
import jax.numpy as jnp
from jax import lax
from jax.experimental import pallas as pl
from jax.experimental.pallas import tpu as pltpu
from jax.experimental.pallas import tpu_sc as plsc

N = 10000
E = 320000
D = 128
NPAD = 10240
NC, NS = 2, 16
NW = NC * NS
EPW = E // NW
K = 80
NCHUNK = EPW // K
ROWS_PER_SUB = NPAD // NS

_mesh = plsc.VectorSubcoreMesh(core_axis_name="c", subcore_axis_name="s")


@functools.partial(
    pl.kernel,
    out_type=jax.ShapeDtypeStruct((NC, NPAD, 16), jnp.float32),
    mesh=_mesh,
    scratch_types=[
        pltpu.VMEM((K,), jnp.int32),
        pltpu.VMEM((K, 16), jnp.float32),
        pltpu.VMEM((ROWS_PER_SUB, 16), jnp.float32),
        pltpu.VMEM_SHARED((NPAD, 16), jnp.float32),
    ],
)
def _deg_kernel(dst_hbm, deg_out, idx_v, ones_v, zb_v, degsh):
    c = lax.axis_index("c")
    s = lax.axis_index("s")
    wid = c * NS + s

    def fill_ones(i, _):
        ones_v[i] = jnp.ones((16,), jnp.float32)
        return _

    lax.fori_loop(0, K, fill_ones, None)

    def fill_z(i, _):
        zb_v[i] = jnp.zeros((16,), jnp.float32)
        return _

    lax.fori_loop(0, ROWS_PER_SUB, fill_z, None)

    pltpu.sync_copy(zb_v, degsh.at[pl.ds(s * ROWS_PER_SUB, ROWS_PER_SUB)])
    plsc.subcore_barrier()

    def body(i, _):
        base = wid * EPW + i * K
        pltpu.sync_copy(dst_hbm.at[pl.ds(base, K)], idx_v)
        pltpu.sync_copy(ones_v, degsh.at[idx_v], add=True)
        return _

    lax.fori_loop(0, NCHUNK, body, None)
    plsc.subcore_barrier()
    pltpu.sync_copy(
        degsh.at[pl.ds(s * ROWS_PER_SUB, ROWS_PER_SUB)],
        deg_out.at[c, pl.ds(s * ROWS_PER_SUB, ROWS_PER_SUB)],
    )


@functools.partial(
    pl.kernel,
    out_type=jax.ShapeDtypeStruct((NC, NPAD, D), jnp.float32),
    mesh=_mesh,
    scratch_types=[
        pltpu.VMEM((K,), jnp.int32),
        pltpu.VMEM((K,), jnp.int32),
        pltpu.VMEM((K, D), jnp.float32),
        pltpu.VMEM((64, D), jnp.float32),
        pltpu.VMEM_SHARED((NPAD, D), jnp.float32),
        pltpu.SemaphoreType.DMA,
    ],
)
def _agg_kernel(h_hbm, src_hbm, dst_hbm, out_hbm, idxs_v, idxd_v, rows_v, zb_v, aggsh, sem):
    c = lax.axis_index("c")
    s = lax.axis_index("s")
    wid = c * NS + s

    def fill_z(i, _):
        for j in range(D // 16):
            zb_v[i, pl.ds(j * 16, 16)] = jnp.zeros((16,), jnp.float32)
        return _

    lax.fori_loop(0, 64, fill_z, None)

    def zcopy(j, _):
        pltpu.sync_copy(zb_v, aggsh.at[pl.ds(s * ROWS_PER_SUB + j * 64, 64)])
        return _

    lax.fori_loop(0, ROWS_PER_SUB // 64, zcopy, None)
    plsc.subcore_barrier()

    def body(i, _):
        base = wid * EPW + i * K
        pltpu.sync_copy(src_hbm.at[pl.ds(base, K)], idxs_v)
        pltpu.sync_copy(dst_hbm.at[pl.ds(base, K)], idxd_v)
        pltpu.async_copy(h_hbm.at[idxs_v], rows_v, sem).wait()
        pltpu.sync_copy(rows_v, aggsh.at[idxd_v], add=True)
        return _

    lax.fori_loop(0, NCHUNK, body, None)
    plsc.subcore_barrier()
    pltpu.sync_copy(
        aggsh.at[pl.ds(s * ROWS_PER_SUB, ROWS_PER_SUB)],
        out_hbm.at[c, pl.ds(s * ROWS_PER_SUB, ROWS_PER_SUB)],
    )


BLK = 1024


def _dinv_from(da, db):
    deg = da[:, 0:1] + db[:, 0:1]
    return jnp.where(deg > 0, lax.rsqrt(deg), 0.0)


def _tc1_body(x_ref, w_ref, da_ref, db_ref, o_ref):
    dinv = _dinv_from(da_ref[...], db_ref[...])
    o_ref[...] = jnp.dot(x_ref[...], w_ref[...], preferred_element_type=jnp.float32) * dinv


def _tc2_body(agg_ref, da_ref, db_ref, b_ref, w_ref, o_ref):
    dinv = _dinv_from(da_ref[...], db_ref[...])
    t = (agg_ref[0] + agg_ref[1]) * dinv + b_ref[...]
    t = jnp.maximum(t, 0.0)
    o_ref[...] = jnp.dot(t, w_ref[...], preferred_element_type=jnp.float32) * dinv


def _tc3_body(agg_ref, da_ref, db_ref, b_ref, o_ref):
    dinv = _dinv_from(da_ref[...], db_ref[...])
    o_ref[...] = (agg_ref[0] + agg_ref[1]) * dinv + b_ref[...]


_deg_spec = pl.BlockSpec((BLK, 16), lambda i: (i, 0))
_row_spec = pl.BlockSpec((BLK, D), lambda i: (i, 0))
_agg_spec = pl.BlockSpec((NC, BLK, D), lambda i: (0, i, 0))
_w_spec = pl.BlockSpec((D, D), lambda i: (0, 0))
_b_spec = pl.BlockSpec((1, D), lambda i: (0, 0))

_tc1 = pl.pallas_call(
    _tc1_body,
    grid=(NPAD // BLK,),
    in_specs=[_row_spec, _w_spec, _deg_spec, _deg_spec],
    out_specs=_row_spec,
    out_shape=jax.ShapeDtypeStruct((NPAD, D), jnp.float32),
)

_tc2 = pl.pallas_call(
    _tc2_body,
    grid=(NPAD // BLK,),
    in_specs=[_agg_spec, _deg_spec, _deg_spec, _b_spec, _w_spec],
    out_specs=_row_spec,
    out_shape=jax.ShapeDtypeStruct((NPAD, D), jnp.float32),
)

_tc3 = pl.pallas_call(
    _tc3_body,
    grid=(NPAD // BLK,),
    in_specs=[_agg_spec, _deg_spec, _deg_spec, _b_spec],
    out_specs=_row_spec,
    out_shape=jax.ShapeDtypeStruct((NPAD, D), jnp.float32),
)


@jax.jit
def kernel(x, edge_index, W1, b1, W2, b2):
    src = edge_index[0].astype(jnp.int32)
    dst = edge_index[1].astype(jnp.int32)
    xp = jnp.zeros((NPAD, D), jnp.float32).at[:N].set(x)

    deg = _deg_kernel(dst)
    da, db = deg[0], deg[1]

    h1 = _tc1(xp, W1, da, db)
    agg1 = _agg_kernel(h1, src, dst)
    h2 = _tc2(agg1, da, db, b1.reshape(1, D), W2)
    agg2 = _agg_kernel(h2, src, dst)
    out = _tc3(agg2, da, db, b2.reshape(1, D))
    return out[:N]


# speedup vs baseline: 10.2101x; 10.2101x over previous
import functools

import jax
import jax.numpy as jnp
from jax import lax
from jax.experimental import pallas as pl
from jax.experimental.pallas import tpu as pltpu
from jax.experimental.pallas import tpu_sc as plsc

N = 10000
E = 320000
D = 128
NPAD = 10240
NC, NS = 2, 16
NW = NC * NS
EPW = E // NW
K = 80
NCHUNK = EPW // K
RPS = NPAD // NS

_mesh = plsc.VectorSubcoreMesh(core_axis_name="c", subcore_axis_name="s")


@functools.partial(
    pl.kernel,
    out_type=jax.ShapeDtypeStruct((NC * NPAD, D), jnp.float32),
    mesh=_mesh,
    scratch_types=[
        pltpu.VMEM((K,), jnp.int32),
        pltpu.VMEM((K, D), jnp.float32),
        pltpu.VMEM((64, D), jnp.float32),
        pltpu.VMEM_SHARED((NPAD, D), jnp.float32),
    ],
)
def _deg_kernel(dst_hbm, zeros_hbm, ones_hbm, deg_out, idx_v, ones_v, zb_v, sh):
    c = lax.axis_index("c")
    s = lax.axis_index("s")
    wid = c * NS + s
    pltpu.sync_copy(zeros_hbm, zb_v)
    pltpu.sync_copy(ones_hbm, ones_v)

    def zcopy(j, _):
        pltpu.sync_copy(zb_v, sh.at[pl.ds(s * RPS + j * 64, 64)])
        return _

    lax.fori_loop(0, RPS // 64, zcopy, None)
    plsc.subcore_barrier()

    def body(i, _):
        base = wid * EPW + i * K
        pltpu.sync_copy(dst_hbm.at[pl.ds(base, K)], idx_v)
        pltpu.sync_copy(ones_v, sh.at[idx_v], add=True)
        return _

    lax.fori_loop(0, NCHUNK, body, None)
    plsc.subcore_barrier()
    pltpu.sync_copy(
        sh.at[pl.ds(s * RPS, RPS)],
        deg_out.at[pl.ds(c * NPAD + s * RPS, RPS)],
    )


@functools.partial(
    pl.kernel,
    out_type=jax.ShapeDtypeStruct((NC * NPAD, D), jnp.float32),
    mesh=_mesh,
    scratch_types=[
        pltpu.VMEM((K,), jnp.int32),
        pltpu.VMEM((K,), jnp.int32),
        pltpu.VMEM((K, D), jnp.float32),
        pltpu.VMEM((64, D), jnp.float32),
        pltpu.VMEM_SHARED((NPAD, D), jnp.float32),
        pltpu.SemaphoreType.DMA,
    ],
)
def _agg_kernel(h_hbm, src_hbm, dst_hbm, zeros_hbm, out_hbm,
                idxs_v, idxd_v, rows_v, zb_v, sh, sem):
    c = lax.axis_index("c")
    s = lax.axis_index("s")
    wid = c * NS + s
    pltpu.sync_copy(zeros_hbm, zb_v)

    def zcopy(j, _):
        pltpu.sync_copy(zb_v, sh.at[pl.ds(s * RPS + j * 64, 64)])
        return _

    lax.fori_loop(0, RPS // 64, zcopy, None)
    plsc.subcore_barrier()

    def body(i, _):
        base = wid * EPW + i * K
        pltpu.sync_copy(src_hbm.at[pl.ds(base, K)], idxs_v)
        pltpu.sync_copy(dst_hbm.at[pl.ds(base, K)], idxd_v)
        pltpu.async_copy(h_hbm.at[idxs_v], rows_v, sem).wait()
        pltpu.sync_copy(rows_v, sh.at[idxd_v], add=True)
        return _

    lax.fori_loop(0, NCHUNK, body, None)
    plsc.subcore_barrier()
    pltpu.sync_copy(
        sh.at[pl.ds(s * RPS, RPS)],
        out_hbm.at[pl.ds(c * NPAD + s * RPS, RPS)],
    )


BLK = 1024


def _dinv_from(da, db):
    deg = da[:, 0:1] + db[:, 0:1]
    return jnp.where(deg > 0, lax.rsqrt(deg), 0.0)


def _tc1_body(x_ref, w_ref, da_ref, db_ref, o_ref):
    dinv = _dinv_from(da_ref[...], db_ref[...])
    o_ref[...] = jnp.dot(x_ref[...], w_ref[...], preferred_element_type=jnp.float32) * dinv


def _tc2_body(aggA_ref, aggB_ref, da_ref, db_ref, b_ref, w_ref, o_ref):
    dinv = _dinv_from(da_ref[...], db_ref[...])
    t = (aggA_ref[...] + aggB_ref[...]) * dinv + b_ref[...]
    t = jnp.maximum(t, 0.0)
    o_ref[...] = jnp.dot(t, w_ref[...], preferred_element_type=jnp.float32) * dinv


def _tc3_body(aggA_ref, aggB_ref, da_ref, db_ref, b_ref, o_ref):
    dinv = _dinv_from(da_ref[...], db_ref[...])
    o_ref[...] = (aggA_ref[...] + aggB_ref[...]) * dinv + b_ref[...]


_row_spec = pl.BlockSpec((BLK, D), lambda i: (i, 0))
_w_spec = pl.BlockSpec((D, D), lambda i: (0, 0))
_b_spec = pl.BlockSpec((1, D), lambda i: (0, 0))

_tc1 = pl.pallas_call(
    _tc1_body,
    grid=(NPAD // BLK,),
    in_specs=[_row_spec, _w_spec, _row_spec, _row_spec],
    out_specs=_row_spec,
    out_shape=jax.ShapeDtypeStruct((NPAD, D), jnp.float32),
)

_tc2 = pl.pallas_call(
    _tc2_body,
    grid=(NPAD // BLK,),
    in_specs=[_row_spec, _row_spec, _row_spec, _row_spec, _b_spec, _w_spec],
    out_specs=_row_spec,
    out_shape=jax.ShapeDtypeStruct((NPAD, D), jnp.float32),
)

_tc3 = pl.pallas_call(
    _tc3_body,
    grid=(NPAD // BLK,),
    in_specs=[_row_spec, _row_spec, _row_spec, _row_spec, _b_spec],
    out_specs=_row_spec,
    out_shape=jax.ShapeDtypeStruct((NPAD, D), jnp.float32),
)


@jax.jit
def kernel(x, edge_index, W1, b1, W2, b2):
    src = edge_index[0].astype(jnp.int32)
    dst = edge_index[1].astype(jnp.int32)
    xp = jnp.zeros((NPAD, D), jnp.float32).at[:N].set(x)
    zeros64 = jnp.zeros((64, D), jnp.float32)
    onesK = jnp.ones((K, D), jnp.float32)

    deg = _deg_kernel(dst, zeros64, onesK)
    da, db = deg[:NPAD], deg[NPAD:]

    h1 = _tc1(xp, W1, da, db)
    agg1 = _agg_kernel(h1, src, dst, zeros64)
    h2 = _tc2(agg1[:NPAD], agg1[NPAD:], da, db, b1.reshape(1, D), W2)
    agg2 = _agg_kernel(h2, src, dst, zeros64)
    out = _tc3(agg2[:NPAD], agg2[NPAD:], da, db, b2.reshape(1, D))
    return out[:N]


# speedup vs baseline: 15.9938x; 1.5665x over previous
import functools

import jax
import jax.numpy as jnp
from jax import lax
from jax.experimental import pallas as pl
from jax.experimental.pallas import tpu as pltpu
from jax.experimental.pallas import tpu_sc as plsc

N = 10000
E = 320000
D = 128
NPAD = 10240
NC, NS = 2, 16
NW = NC * NS
EPW = E // NW
K = 80
NCHUNK = EPW // K
RPS = NPAD // NS

_mesh = plsc.VectorSubcoreMesh(core_axis_name="c", subcore_axis_name="s")


@functools.partial(
    pl.kernel,
    out_type=jax.ShapeDtypeStruct((NC * NPAD, D), jnp.float32),
    mesh=_mesh,
    scratch_types=[
        pltpu.VMEM((K,), jnp.int32),
        pltpu.VMEM((K,), jnp.int32),
        pltpu.VMEM((K, D), jnp.float32),
        pltpu.VMEM((64, D), jnp.float32),
        pltpu.VMEM_SHARED((NPAD, D), jnp.float32),
        pltpu.SemaphoreType.DMA,
        pltpu.SemaphoreType.DMA,
    ],
)
def _deg_kernel(dst_hbm, zeros_hbm, ones_hbm, deg_out,
                idx_a, idx_b, ones_v, zb_v, sh, sem_a, sem_b):
    c = lax.axis_index("c")
    s = lax.axis_index("s")
    wid = c * NS + s
    base0 = wid * EPW
    pltpu.sync_copy(zeros_hbm, zb_v)
    pltpu.sync_copy(ones_hbm, ones_v)

    def zcopy(j, _):
        pltpu.sync_copy(zb_v, sh.at[pl.ds(s * RPS + j * 64, 64)])
        return _

    lax.fori_loop(0, RPS // 64, zcopy, None)
    plsc.subcore_barrier()

    pltpu.async_copy(dst_hbm.at[pl.ds(base0, K)], idx_a, sem_a)

    def body(i, _):
        c1 = 2 * i + 1
        c2 = 2 * i + 2
        pltpu.async_copy(dst_hbm.at[pl.ds(base0 + c1 * K, K)], idx_b, sem_b)
        pltpu.make_async_copy(dst_hbm.at[pl.ds(base0, K)], idx_a, sem_a).wait()
        pltpu.sync_copy(ones_v, sh.at[idx_a], add=True)
        pltpu.async_copy(dst_hbm.at[pl.ds(base0 + c2 * K, K)], idx_a, sem_a)
        pltpu.make_async_copy(dst_hbm.at[pl.ds(base0, K)], idx_b, sem_b).wait()
        pltpu.sync_copy(ones_v, sh.at[idx_b], add=True)
        return _

    lax.fori_loop(0, (NCHUNK - 1) // 2, body, None)
    pltpu.make_async_copy(dst_hbm.at[pl.ds(base0, K)], idx_a, sem_a).wait()
    pltpu.sync_copy(ones_v, sh.at[idx_a], add=True)
    plsc.subcore_barrier()
    pltpu.sync_copy(
        sh.at[pl.ds(s * RPS, RPS)],
        deg_out.at[pl.ds(c * NPAD + s * RPS, RPS)],
    )


@functools.partial(
    pl.kernel,
    out_type=jax.ShapeDtypeStruct((NC * NPAD, D), jnp.float32),
    mesh=_mesh,
    scratch_types=[
        pltpu.VMEM((K,), jnp.int32),
        pltpu.VMEM((K,), jnp.int32),
        pltpu.VMEM((K, D), jnp.float32),
        pltpu.VMEM((K,), jnp.int32),
        pltpu.VMEM((K,), jnp.int32),
        pltpu.VMEM((K, D), jnp.float32),
        pltpu.VMEM((64, D), jnp.float32),
        pltpu.VMEM_SHARED((NPAD, D), jnp.float32),
        pltpu.SemaphoreType.DMA,
        pltpu.SemaphoreType.DMA,
    ],
)
def _agg_kernel(h_hbm, src_hbm, dst_hbm, zeros_hbm, out_hbm,
                idxs_a, idxd_a, rows_a, idxs_b, idxd_b, rows_b,
                zb_v, sh, sem_a, sem_b):
    c = lax.axis_index("c")
    s = lax.axis_index("s")
    wid = c * NS + s
    base0 = wid * EPW
    pltpu.sync_copy(zeros_hbm, zb_v)

    def zcopy(j, _):
        pltpu.sync_copy(zb_v, sh.at[pl.ds(s * RPS + j * 64, 64)])
        return _

    lax.fori_loop(0, RPS // 64, zcopy, None)
    plsc.subcore_barrier()

    pltpu.sync_copy(src_hbm.at[pl.ds(base0, K)], idxs_a)
    pltpu.sync_copy(dst_hbm.at[pl.ds(base0, K)], idxd_a)
    pltpu.async_copy(h_hbm.at[idxs_a], rows_a, sem_a)

    def body(i, _):
        c1 = 2 * i + 1
        c2 = 2 * i + 2
        pltpu.sync_copy(src_hbm.at[pl.ds(base0 + c1 * K, K)], idxs_b)
        pltpu.sync_copy(dst_hbm.at[pl.ds(base0 + c1 * K, K)], idxd_b)
        pltpu.async_copy(h_hbm.at[idxs_b], rows_b, sem_b)
        pltpu.make_async_copy(h_hbm.at[idxs_a], rows_a, sem_a).wait()
        pltpu.sync_copy(rows_a, sh.at[idxd_a], add=True)
        pltpu.sync_copy(src_hbm.at[pl.ds(base0 + c2 * K, K)], idxs_a)
        pltpu.sync_copy(dst_hbm.at[pl.ds(base0 + c2 * K, K)], idxd_a)
        pltpu.async_copy(h_hbm.at[idxs_a], rows_a, sem_a)
        pltpu.make_async_copy(h_hbm.at[idxs_b], rows_b, sem_b).wait()
        pltpu.sync_copy(rows_b, sh.at[idxd_b], add=True)
        return _

    lax.fori_loop(0, (NCHUNK - 1) // 2, body, None)
    pltpu.make_async_copy(h_hbm.at[idxs_a], rows_a, sem_a).wait()
    pltpu.sync_copy(rows_a, sh.at[idxd_a], add=True)
    plsc.subcore_barrier()
    pltpu.sync_copy(
        sh.at[pl.ds(s * RPS, RPS)],
        out_hbm.at[pl.ds(c * NPAD + s * RPS, RPS)],
    )


BLK = 1024


def _dinv_from(da, db):
    deg = da[:, 0:1] + db[:, 0:1]
    return jnp.where(deg > 0, lax.rsqrt(deg), 0.0)


def _tc1_body(x_ref, w_ref, da_ref, db_ref, o_ref):
    dinv = _dinv_from(da_ref[...], db_ref[...])
    o_ref[...] = jnp.dot(x_ref[...], w_ref[...], preferred_element_type=jnp.float32) * dinv


def _tc2_body(aggA_ref, aggB_ref, da_ref, db_ref, b_ref, w_ref, o_ref):
    dinv = _dinv_from(da_ref[...], db_ref[...])
    t = (aggA_ref[...] + aggB_ref[...]) * dinv + b_ref[...]
    t = jnp.maximum(t, 0.0)
    o_ref[...] = jnp.dot(t, w_ref[...], preferred_element_type=jnp.float32) * dinv


def _tc3_body(aggA_ref, aggB_ref, da_ref, db_ref, b_ref, o_ref):
    dinv = _dinv_from(da_ref[...], db_ref[...])
    o_ref[...] = (aggA_ref[...] + aggB_ref[...]) * dinv + b_ref[...]


_row_spec = pl.BlockSpec((BLK, D), lambda i: (i, 0))
_w_spec = pl.BlockSpec((D, D), lambda i: (0, 0))
_b_spec = pl.BlockSpec((1, D), lambda i: (0, 0))

_tc1 = pl.pallas_call(
    _tc1_body,
    grid=(NPAD // BLK,),
    in_specs=[_row_spec, _w_spec, _row_spec, _row_spec],
    out_specs=_row_spec,
    out_shape=jax.ShapeDtypeStruct((NPAD, D), jnp.float32),
)

_tc2 = pl.pallas_call(
    _tc2_body,
    grid=(NPAD // BLK,),
    in_specs=[_row_spec, _row_spec, _row_spec, _row_spec, _b_spec, _w_spec],
    out_specs=_row_spec,
    out_shape=jax.ShapeDtypeStruct((NPAD, D), jnp.float32),
)

_tc3 = pl.pallas_call(
    _tc3_body,
    grid=(NPAD // BLK,),
    in_specs=[_row_spec, _row_spec, _row_spec, _row_spec, _b_spec],
    out_specs=_row_spec,
    out_shape=jax.ShapeDtypeStruct((NPAD, D), jnp.float32),
)


@jax.jit
def kernel(x, edge_index, W1, b1, W2, b2):
    src = edge_index[0].astype(jnp.int32)
    dst = edge_index[1].astype(jnp.int32)
    xp = jnp.zeros((NPAD, D), jnp.float32).at[:N].set(x)
    zeros64 = jnp.zeros((64, D), jnp.float32)
    onesK = jnp.ones((K, D), jnp.float32)

    deg = _deg_kernel(dst, zeros64, onesK)
    da, db = deg[:NPAD], deg[NPAD:]

    h1 = _tc1(xp, W1, da, db)
    agg1 = _agg_kernel(h1, src, dst, zeros64)
    h2 = _tc2(agg1[:NPAD], agg1[NPAD:], da, db, b1.reshape(1, D), W2)
    agg2 = _agg_kernel(h2, src, dst, zeros64)
    out = _tc3(agg2[:NPAD], agg2[NPAD:], da, db, b2.reshape(1, D))
    return out[:N]


# speedup vs baseline: 18.3778x; 1.1491x over previous
import functools

import jax
import jax.numpy as jnp
from jax import lax
from jax.experimental import pallas as pl
from jax.experimental.pallas import tpu as pltpu
from jax.experimental.pallas import tpu_sc as plsc

N = 10000
E = 320000
D = 128
NPAD = 10240
NC, NS = 2, 16
NW = NC * NS
EPW = E // NW
K = 80
NCHUNK = EPW // K
RPS = NPAD // NS

_mesh = plsc.VectorSubcoreMesh(core_axis_name="c", subcore_axis_name="s")


@functools.partial(
    pl.kernel,
    out_type=jax.ShapeDtypeStruct((NC * NPAD, D), jnp.float32),
    mesh=_mesh,
    scratch_types=[
        pltpu.VMEM((K,), jnp.int32),
        pltpu.VMEM((K,), jnp.int32),
        pltpu.VMEM((K, D), jnp.float32),
        pltpu.VMEM((64, D), jnp.float32),
        pltpu.VMEM_SHARED((NPAD, D), jnp.float32),
        pltpu.SemaphoreType.DMA,
        pltpu.SemaphoreType.DMA,
    ],
)
def _deg_kernel(dst_hbm, zeros_hbm, ones_hbm, deg_out,
                idx_a, idx_b, ones_v, zb_v, sh, sem_a, sem_b):
    c = lax.axis_index("c")
    s = lax.axis_index("s")
    wid = c * NS + s
    base0 = wid * EPW
    pltpu.sync_copy(zeros_hbm, zb_v)
    pltpu.sync_copy(ones_hbm, ones_v)

    def zcopy(j, _):
        pltpu.sync_copy(zb_v, sh.at[pl.ds(s * RPS + j * 64, 64)])
        return _

    lax.fori_loop(0, RPS // 64, zcopy, None)
    plsc.subcore_barrier()

    pltpu.async_copy(dst_hbm.at[pl.ds(base0, K)], idx_a, sem_a)

    def body(i, _):
        c1 = 2 * i + 1
        c2 = 2 * i + 2
        pltpu.async_copy(dst_hbm.at[pl.ds(base0 + c1 * K, K)], idx_b, sem_b)
        pltpu.make_async_copy(dst_hbm.at[pl.ds(base0, K)], idx_a, sem_a).wait()
        pltpu.sync_copy(ones_v, sh.at[idx_a], add=True)
        pltpu.async_copy(dst_hbm.at[pl.ds(base0 + c2 * K, K)], idx_a, sem_a)
        pltpu.make_async_copy(dst_hbm.at[pl.ds(base0, K)], idx_b, sem_b).wait()
        pltpu.sync_copy(ones_v, sh.at[idx_b], add=True)
        return _

    lax.fori_loop(0, (NCHUNK - 1) // 2, body, None)
    pltpu.make_async_copy(dst_hbm.at[pl.ds(base0, K)], idx_a, sem_a).wait()
    pltpu.sync_copy(ones_v, sh.at[idx_a], add=True)
    plsc.subcore_barrier()
    pltpu.sync_copy(
        sh.at[pl.ds(s * RPS, RPS)],
        deg_out.at[pl.ds(c * NPAD + s * RPS, RPS)],
    )


@functools.partial(
    pl.kernel,
    out_type=jax.ShapeDtypeStruct((NC * NPAD, D), jnp.float32),
    mesh=_mesh,
    scratch_types=[
        pltpu.VMEM((K,), jnp.int32),
        pltpu.VMEM((K,), jnp.int32),
        pltpu.VMEM((K, D), jnp.float32),
        pltpu.VMEM((K,), jnp.int32),
        pltpu.VMEM((K,), jnp.int32),
        pltpu.VMEM((K, D), jnp.float32),
        pltpu.VMEM((64, D), jnp.float32),
        pltpu.VMEM_SHARED((NPAD, D), jnp.float32),
        pltpu.SemaphoreType.DMA,
        pltpu.SemaphoreType.DMA,
        pltpu.SemaphoreType.DMA,
        pltpu.SemaphoreType.DMA,
    ],
)
def _agg_kernel(h_hbm, src_hbm, dst_hbm, zeros_hbm, out_hbm,
                idxs_a, idxd_a, rows_a, idxs_b, idxd_b, rows_b,
                zb_v, sh, sem_a, sem_b, sem_ia, sem_ib):
    c = lax.axis_index("c")
    s = lax.axis_index("s")
    wid = c * NS + s
    base0 = wid * EPW
    pltpu.sync_copy(zeros_hbm, zb_v)

    def zcopy(j, _):
        pltpu.sync_copy(zb_v, sh.at[pl.ds(s * RPS + j * 64, 64)])
        return _

    lax.fori_loop(0, RPS // 64, zcopy, None)
    plsc.subcore_barrier()

    def idx_start(c_idx, idxs, idxd, sem):
        pltpu.async_copy(src_hbm.at[pl.ds(base0 + c_idx * K, K)], idxs, sem)
        pltpu.async_copy(dst_hbm.at[pl.ds(base0 + c_idx * K, K)], idxd, sem)

    def idx_wait(idxs, idxd, sem):
        pltpu.make_async_copy(src_hbm.at[pl.ds(base0, K)], idxs, sem).wait()
        pltpu.make_async_copy(dst_hbm.at[pl.ds(base0, K)], idxd, sem).wait()

    pltpu.sync_copy(src_hbm.at[pl.ds(base0, K)], idxs_a)
    pltpu.sync_copy(dst_hbm.at[pl.ds(base0, K)], idxd_a)
    pltpu.async_copy(h_hbm.at[idxs_a], rows_a, sem_a)
    idx_start(1, idxs_b, idxd_b, sem_ib)

    def body(i, _):
        c2 = 2 * i + 2
        c3 = 2 * i + 3
        idx_wait(idxs_b, idxd_b, sem_ib)
        pltpu.async_copy(h_hbm.at[idxs_b], rows_b, sem_b)
        pltpu.make_async_copy(h_hbm.at[idxs_a], rows_a, sem_a).wait()
        pltpu.sync_copy(rows_a, sh.at[idxd_a], add=True)
        idx_start(c2, idxs_a, idxd_a, sem_ia)
        pltpu.make_async_copy(h_hbm.at[idxs_b], rows_b, sem_b).wait()
        pltpu.sync_copy(rows_b, sh.at[idxd_b], add=True)
        idx_start(jnp.minimum(c3, NCHUNK - 1), idxs_b, idxd_b, sem_ib)
        idx_wait(idxs_a, idxd_a, sem_ia)
        pltpu.async_copy(h_hbm.at[idxs_a], rows_a, sem_a)
        return _

    lax.fori_loop(0, (NCHUNK - 1) // 2, body, None)
    idx_wait(idxs_b, idxd_b, sem_ib)
    pltpu.make_async_copy(h_hbm.at[idxs_a], rows_a, sem_a).wait()
    pltpu.sync_copy(rows_a, sh.at[idxd_a], add=True)
    plsc.subcore_barrier()
    pltpu.sync_copy(
        sh.at[pl.ds(s * RPS, RPS)],
        out_hbm.at[pl.ds(c * NPAD + s * RPS, RPS)],
    )


BLK = 1024


def _dinv_from(da, db):
    deg = da[:, 0:1] + db[:, 0:1]
    return jnp.where(deg > 0, lax.rsqrt(deg), 0.0)


def _tc1_body(x_ref, w_ref, da_ref, db_ref, o_ref):
    dinv = _dinv_from(da_ref[...], db_ref[...])
    o_ref[...] = jnp.dot(x_ref[...], w_ref[...], preferred_element_type=jnp.float32) * dinv


def _tc2_body(aggA_ref, aggB_ref, da_ref, db_ref, b_ref, w_ref, o_ref):
    dinv = _dinv_from(da_ref[...], db_ref[...])
    t = (aggA_ref[...] + aggB_ref[...]) * dinv + b_ref[...]
    t = jnp.maximum(t, 0.0)
    o_ref[...] = jnp.dot(t, w_ref[...], preferred_element_type=jnp.float32) * dinv


def _tc3_body(aggA_ref, aggB_ref, da_ref, db_ref, b_ref, o_ref):
    dinv = _dinv_from(da_ref[...], db_ref[...])
    o_ref[...] = (aggA_ref[...] + aggB_ref[...]) * dinv + b_ref[...]


_row_spec = pl.BlockSpec((BLK, D), lambda i: (i, 0))
_w_spec = pl.BlockSpec((D, D), lambda i: (0, 0))
_b_spec = pl.BlockSpec((1, D), lambda i: (0, 0))

_tc1 = pl.pallas_call(
    _tc1_body,
    grid=(NPAD // BLK,),
    in_specs=[_row_spec, _w_spec, _row_spec, _row_spec],
    out_specs=_row_spec,
    out_shape=jax.ShapeDtypeStruct((NPAD, D), jnp.float32),
)

_tc2 = pl.pallas_call(
    _tc2_body,
    grid=(NPAD // BLK,),
    in_specs=[_row_spec, _row_spec, _row_spec, _row_spec, _b_spec, _w_spec],
    out_specs=_row_spec,
    out_shape=jax.ShapeDtypeStruct((NPAD, D), jnp.float32),
)

_tc3 = pl.pallas_call(
    _tc3_body,
    grid=(NPAD // BLK,),
    in_specs=[_row_spec, _row_spec, _row_spec, _row_spec, _b_spec],
    out_specs=_row_spec,
    out_shape=jax.ShapeDtypeStruct((NPAD, D), jnp.float32),
)


@jax.jit
def kernel(x, edge_index, W1, b1, W2, b2):
    src = edge_index[0].astype(jnp.int32)
    dst = edge_index[1].astype(jnp.int32)
    xp = jnp.zeros((NPAD, D), jnp.float32).at[:N].set(x)
    zeros64 = jnp.zeros((64, D), jnp.float32)
    onesK = jnp.ones((K, D), jnp.float32)

    deg = _deg_kernel(dst, zeros64, onesK)
    da, db = deg[:NPAD], deg[NPAD:]

    h1 = _tc1(xp, W1, da, db)
    agg1 = _agg_kernel(h1, src, dst, zeros64)
    h2 = _tc2(agg1[:NPAD], agg1[NPAD:], da, db, b1.reshape(1, D), W2)
    agg2 = _agg_kernel(h2, src, dst, zeros64)
    out = _tc3(agg2[:NPAD], agg2[NPAD:], da, db, b2.reshape(1, D))
    return out[:N]
